# baseline (speedup 1.0000x reference)
import functools

import jax
import jax.numpy as jnp
from jax import lax
from jax.experimental import pallas as pl
from jax.experimental.pallas import tpu as pltpu
from jax.experimental.pallas import tpu_sc as plsc

_F = 128
_BOARD = 15
_P = _BOARD * _BOARD
_PP = 256
_PCODE_DIM = 2380
_EMBED_DIM = 2 * (_PCODE_DIM + 1)
_BATCH = 1024
_NTILES = 32
_BPT = _BATCH // _NTILES
_HROWS = 464


def _sc_embed(inp, offp, valid16, hs_idx, hb_idx, tbl_small, tbl_big, nb):
    bpt = nb // _NTILES
    mesh = plsc.VectorSubcoreMesh(
        core_axis_name="c", subcore_axis_name="s", num_cores=2, num_subcores=16
    )

    @functools.partial(
        pl.kernel,
        out_type=(
            jax.ShapeDtypeStruct((nb, _P, _F), jnp.float32),
            jax.ShapeDtypeStruct((_HROWS, _F), jnp.float32),
        ),
        mesh=mesh,
        scratch_types=[
            pltpu.VMEM((4, _PP), jnp.int32),
            pltpu.VMEM((_PP,), jnp.int32),
            pltpu.VMEM((16,), jnp.int32),
            pltpu.VMEM((_PP,), jnp.int32),
            pltpu.VMEM((_PP,), jnp.int32),
            pltpu.VMEM((_PP,), jnp.int32),
            pltpu.VMEM((_PP,), jnp.int32),
            pltpu.VMEM((2, _PP, 16), jnp.float32),
            pltpu.VMEM((128, _F), jnp.float32),
            pltpu.VMEM((64, _F), jnp.float32),
            pltpu.VMEM((64, _F), jnp.float32),
            pltpu.VMEM((64, _F), jnp.float32),
            pltpu.VMEM((64, _F), jnp.float32),
            pltpu.VMEM((16,), jnp.int32),
            pltpu.VMEM((16,), jnp.int32),
            pltpu.SemaphoreType.DMA,
        ],
    )
    def k(inp_h, offp_h, valid_h, hsi_h, hbi_h, tbls_h, tblb_h, out_h, hout_h,
          ibuf, offb, vb, i0e, i0o, i1e, i1o, wexp, acc,
          gA0, gB0, gA1, gB1, hsb, hbb, semG):
        wid = lax.axis_index("s") * 2 + lax.axis_index("c")
        pltpu.sync_copy(offp_h, offb)
        pltpu.sync_copy(valid_h, vb)

        @pl.when(wid < _HROWS // 16)
        def _h_phase():
            pltpu.sync_copy(hsi_h.at[pl.ds(wid * 16, 16)], hsb)
            pltpu.sync_copy(hbi_h.at[pl.ds(wid * 16, 16)], hbb)
            ha = pltpu.async_copy(tbls_h.at[hsb], gA0.at[pl.ds(0, 16)], semG)
            hb = pltpu.async_copy(tblb_h.at[hbb], gB0.at[pl.ds(0, 16)], semG)
            ha.wait()
            hb.wait()
            for r in range(16):
                for kk in range(_F // 16):
                    sl = pl.ds(kk * 16, 16)
                    gA0[r, sl] = gA0[r, sl] + gB0[r, sl]
            pltpu.sync_copy(gA0.at[pl.ds(0, 16)], hout_h.at[pl.ds(wid * 16, 16)])

        gbufs = ((gA0, gB0), (gA1, gB1))
        idx_e = (i0e, i1e)
        idx_o = (i0o, i1o)

        def fire(c, half, q, s):
            sl = pl.ds(half * 128 + q * 64, 64)
            gA, gB = gbufs[s]
            return (pltpu.async_copy(tbls_h.at[idx_e[c].at[sl]], gA, semG),
                    pltpu.async_copy(tblb_h.at[idx_o[c].at[sl]], gB, semG))

        def accum(c, half, q, s):
            gA, gB = gbufs[s]

            def row(r, _):
                w = wexp[c, half * 128 + q * 64 + r, :]
                for kk in range(_F // 16):
                    sl = pl.ds(kk * 16, 16)
                    v = (gA[r, sl] + gB[r, sl]) * w
                    if c == 0:
                        acc[q * 64 + r, sl] = v
                    else:
                        plsc.addupdate(acc.at[q * 64 + r, sl], v)
                return 0

            lax.fori_loop(0, 64, row, 0)

        def per_b(j, _):
            b = wid * bpt + j
            pltpu.sync_copy(inp_h.at[b], ibuf)
            vv = vb[...]
            for i in range(_PP // 16):
                sl = pl.ds(i * 16, 16)
                offv = offb[sl]
                pc0 = ibuf[0, sl]
                pc1 = ibuf[1, sl]
                bd0 = ibuf[2, sl]
                bd1 = ibuf[3, sl]
                e0 = pc0 * vv
                e1 = pc1 * vv + (_PCODE_DIM + 1)
                i0e[sl] = e0
                i0o[sl] = e0 + offv
                i1e[sl] = e1
                i1o[sl] = e1 + offv
                one = jnp.full((16,), 1.0, dtype=jnp.float32)
                zero = jnp.full((16,), 0.0, dtype=jnp.float32)
                w0 = jnp.where(bd0 > 0, zero, one)
                w1 = jnp.where(bd1 > 0, zero, one)
                for l in range(16):
                    wexp[0, i * 16 + l, :] = jnp.full((16,), w0[l], dtype=jnp.float32)
                    wexp[1, i * 16 + l, :] = jnp.full((16,), w1[l], dtype=jnp.float32)
            for half in range(2):
                units = [(c, q) for c in range(2) for q in range(2)]
                pend = [None, None]
                for u, (c, q) in enumerate(units):
                    s = u % 2
                    if pend[s] is not None:
                        pc_, pq_, hh_ = pend[s]
                        for hh in hh_:
                            hh.wait()
                        accum(pc_, pq_[0], pq_[1], s)
                    pend[s] = (c, (half, q), fire(c, half, q, s))
                for s in range(2):
                    pc_, pq_, hh_ = pend[s]
                    for hh in hh_:
                        hh.wait()
                    accum(pc_, pq_[0], pq_[1], s)
                nrows = 128 if half == 0 else _P - 128
                pltpu.sync_copy(acc.at[pl.ds(0, nrows)],
                                out_h.at[b].at[pl.ds(half * 128, nrows)])
            return 0

        lax.fori_loop(0, bpt, per_b, 0)

    return k(inp, offp, valid16, hs_idx, hb_idx, tbl_small, tbl_big)


def _tc_finish(sc_out, board3, h, nb):
    TB = 8

    def body(sc_ref, bd_ref, h_ref, o_ref):
        x = sc_ref[...]
        bd = bd_ref[...]
        hh = h_ref[...]
        m0 = (bd[:, 0, :] > 0).astype(jnp.float32)[..., None]
        m1 = (bd[:, 1, :] > 0).astype(jnp.float32)[..., None]
        y = x + m0 * hh[0] + m1 * hh[1]
        o_ref[...] = jnp.transpose(y, (0, 2, 1))

    return pl.pallas_call(
        body,
        out_shape=jax.ShapeDtypeStruct((nb, _F, _P), jnp.float32),
        grid=(nb // TB,),
        in_specs=[
            pl.BlockSpec((TB, _P, _F), lambda i: (i, 0, 0)),
            pl.BlockSpec((TB, 2, _P), lambda i: (i, 0, 0)),
            pl.BlockSpec((2, _P, _F), lambda i: (0, 0, 0)),
        ],
        out_specs=pl.BlockSpec((TB, _F, _P), lambda i: (i, 0, 0)),
    )(sc_out, board3, h)


def kernel(sparse_feature_input, board_input, sparse_feature_dim,
           pcode_embedding, pcode_outerboard_embedding, offset_map):
    valid = jnp.all(sparse_feature_dim[:, 10:12] == _PCODE_DIM)
    pc = sparse_feature_input[:, 10:12].reshape(_BATCH, 2, _P)
    bd = board_input.reshape(_BATCH, 2, _P)

    npad = _PP - _P
    pad_pc = ((jnp.arange(npad, dtype=jnp.int32) * 97) % _PCODE_DIM)
    pad_pc = jnp.broadcast_to(pad_pc, (_BATCH, 2, npad))
    pad_bd = jnp.ones((_BATCH, 2, npad), jnp.int32)
    inp = jnp.concatenate(
        [jnp.concatenate([pc, pad_pc], axis=2),
         jnp.concatenate([bd, pad_bd], axis=2)], axis=1)

    off_flat = offset_map.reshape(_P)
    pad_off = ((jnp.arange(npad, dtype=jnp.int32) * 31) % 121) * _EMBED_DIM
    offp = jnp.concatenate([off_flat, pad_off])
    valid16 = jnp.full((16,), valid.astype(jnp.int32), dtype=jnp.int32)

    sent = jnp.concatenate([
        jnp.full((_P,), _PCODE_DIM, jnp.int32),
        jnp.full((_P,), 2 * _PCODE_DIM + 1, jnp.int32),
    ])
    hpad = _HROWS - 2 * _P
    hs_idx = jnp.concatenate([sent, (jnp.arange(hpad, dtype=jnp.int32) * 13) % _PCODE_DIM])
    hb_idx = jnp.concatenate([
        sent + jnp.concatenate([off_flat, off_flat]),
        (jnp.arange(hpad, dtype=jnp.int32) * 17) % _EMBED_DIM,
    ])

    nch = 4
    bc = _BATCH // nch
    h = None
    outs = []
    for ch in range(nch):
        sc_out, h_out = _sc_embed(
            inp[ch * bc:(ch + 1) * bc], offp, valid16, hs_idx, hb_idx,
            pcode_embedding, pcode_outerboard_embedding, bc)
        if h is None:
            h = h_out[: 2 * _P].reshape(2, _P, _F)
        outs.append(_tc_finish(sc_out, bd[ch * bc:(ch + 1) * bc], h, bc))
    out = jnp.concatenate(outs, axis=0)
    return out.reshape(_BATCH, _F, _BOARD, _BOARD)


# speedup vs baseline: 1.0654x; 1.0654x over previous
import functools

import jax
import jax.numpy as jnp
from jax import lax
from jax.experimental import pallas as pl
from jax.experimental.pallas import tpu as pltpu
from jax.experimental.pallas import tpu_sc as plsc

_F = 128
_BOARD = 15
_P = _BOARD * _BOARD
_PP = 256
_PCODE_DIM = 2380
_EMBED_DIM = 2 * (_PCODE_DIM + 1)
_BATCH = 1024
_NTILES = 32
_BPT = _BATCH // _NTILES
_HROWS = 464


def _sc_embed(inp, offp, valid16, hs_idx, hb_idx, tbl_small, tbl_big):
    mesh = plsc.VectorSubcoreMesh(
        core_axis_name="c", subcore_axis_name="s", num_cores=2, num_subcores=16
    )

    @functools.partial(
        pl.kernel,
        out_type=(
            jax.ShapeDtypeStruct((_BATCH, _P, _F), jnp.float32),
            jax.ShapeDtypeStruct((_HROWS, _F), jnp.float32),
        ),
        mesh=mesh,
        scratch_types=[
            pltpu.VMEM((4, _PP), jnp.int32),
            pltpu.VMEM((_PP,), jnp.int32),
            pltpu.VMEM((16,), jnp.int32),
            pltpu.VMEM((_PP,), jnp.int32),
            pltpu.VMEM((_PP,), jnp.int32),
            pltpu.VMEM((_PP,), jnp.int32),
            pltpu.VMEM((_PP,), jnp.int32),
            pltpu.VMEM((2, _PP, 16), jnp.float32),
            pltpu.VMEM((128, _F), jnp.float32),
            pltpu.VMEM((64, _F), jnp.float32),
            pltpu.VMEM((64, _F), jnp.float32),
            pltpu.VMEM((64, _F), jnp.float32),
            pltpu.VMEM((64, _F), jnp.float32),
            pltpu.VMEM((16,), jnp.int32),
            pltpu.VMEM((16,), jnp.int32),
            pltpu.SemaphoreType.DMA,
        ],
    )
    def k(inp_h, offp_h, valid_h, hsi_h, hbi_h, tbls_h, tblb_h, out_h, hout_h,
          ibuf, offb, vb, i0e, i0o, i1e, i1o, wexp, acc,
          gA0, gB0, gA1, gB1, hsb, hbb, semG):
        wid = lax.axis_index("s") * 2 + lax.axis_index("c")
        pltpu.sync_copy(offp_h, offb)
        pltpu.sync_copy(valid_h, vb)

        @pl.when(wid < _HROWS // 16)
        def _h_phase():
            pltpu.sync_copy(hsi_h.at[pl.ds(wid * 16, 16)], hsb)
            pltpu.sync_copy(hbi_h.at[pl.ds(wid * 16, 16)], hbb)
            ha = pltpu.async_copy(tbls_h.at[hsb], gA0.at[pl.ds(0, 16)], semG)
            hb = pltpu.async_copy(tblb_h.at[hbb], gB0.at[pl.ds(0, 16)], semG)
            ha.wait()
            hb.wait()
            for r in range(16):
                for kk in range(_F // 16):
                    sl = pl.ds(kk * 16, 16)
                    gA0[r, sl] = gA0[r, sl] + gB0[r, sl]
            pltpu.sync_copy(gA0.at[pl.ds(0, 16)], hout_h.at[pl.ds(wid * 16, 16)])

        gbufs = ((gA0, gB0), (gA1, gB1))
        idx_e = (i0e, i1e)
        idx_o = (i0o, i1o)

        def fire(c, half, q, s):
            sl = pl.ds(half * 128 + q * 64, 64)
            gA, gB = gbufs[s]
            return (pltpu.async_copy(tbls_h.at[idx_e[c].at[sl]], gA, semG),
                    pltpu.async_copy(tblb_h.at[idx_o[c].at[sl]], gB, semG))

        def accum(c, half, q, s):
            gA, gB = gbufs[s]

            def row(r, _):
                w = wexp[c, half * 128 + q * 64 + r, :]
                for kk in range(_F // 16):
                    sl = pl.ds(kk * 16, 16)
                    v = (gA[r, sl] + gB[r, sl]) * w
                    if c == 0:
                        acc[q * 64 + r, sl] = v
                    else:
                        plsc.addupdate(acc.at[q * 64 + r, sl], v)
                return 0

            lax.fori_loop(0, 64, row, 0)

        def per_b(j, _):
            b = wid * _BPT + j
            pltpu.sync_copy(inp_h.at[b], ibuf)
            vv = vb[...]
            for i in range(_PP // 16):
                sl = pl.ds(i * 16, 16)
                offv = offb[sl]
                pc0 = ibuf[0, sl]
                pc1 = ibuf[1, sl]
                e0 = pc0 * vv
                e1 = pc1 * vv + (_PCODE_DIM + 1)
                i0e[sl] = e0
                i0o[sl] = e0 + offv
                i1e[sl] = e1
                i1o[sl] = e1 + offv

            def build_w():
                one = jnp.full((16,), 1.0, dtype=jnp.float32)
                zero = jnp.full((16,), 0.0, dtype=jnp.float32)
                for i in range(_PP // 16):
                    sl = pl.ds(i * 16, 16)
                    w0 = jnp.where(ibuf[2, sl] > 0, zero, one)
                    w1 = jnp.where(ibuf[3, sl] > 0, zero, one)
                    for l in range(16):
                        wexp[0, i * 16 + l, :] = jnp.full((16,), w0[l], dtype=jnp.float32)
                        wexp[1, i * 16 + l, :] = jnp.full((16,), w1[l], dtype=jnp.float32)

            first = [True]
            for half in range(2):
                units = [(c, q) for c in range(2) for q in range(2)]
                pend = [None, None]
                for u, (c, q) in enumerate(units):
                    s = u % 2
                    if pend[s] is not None:
                        pc_, pq_, hh_ = pend[s]
                        for hh in hh_:
                            hh.wait()
                        accum(pc_, pq_[0], pq_[1], s)
                    pend[s] = (c, (half, q), fire(c, half, q, s))
                    if first[0] and u == 1:
                        build_w()
                        first[0] = False
                for s in range(2):
                    pc_, pq_, hh_ = pend[s]
                    for hh in hh_:
                        hh.wait()
                    accum(pc_, pq_[0], pq_[1], s)
                nrows = 128 if half == 0 else _P - 128
                pltpu.sync_copy(acc.at[pl.ds(0, nrows)],
                                out_h.at[b].at[pl.ds(half * 128, nrows)])
            return 0

        lax.fori_loop(0, _BPT, per_b, 0)

    return k(inp, offp, valid16, hs_idx, hb_idx, tbl_small, tbl_big)


def _tc_finish(sc_out, board3, h):
    TB = 8

    def body(sc_ref, bd_ref, h_ref, o_ref):
        x = sc_ref[...]
        bd = bd_ref[...]
        hh = h_ref[...]
        m0 = (bd[:, 0, :] > 0).astype(jnp.float32)[..., None]
        m1 = (bd[:, 1, :] > 0).astype(jnp.float32)[..., None]
        y = x + m0 * hh[0] + m1 * hh[1]
        o_ref[...] = jnp.transpose(y, (0, 2, 1))

    return pl.pallas_call(
        body,
        out_shape=jax.ShapeDtypeStruct((_BATCH, _F, _P), jnp.float32),
        grid=(_BATCH // TB,),
        in_specs=[
            pl.BlockSpec((TB, _P, _F), lambda i: (i, 0, 0)),
            pl.BlockSpec((TB, 2, _P), lambda i: (i, 0, 0)),
            pl.BlockSpec((2, _P, _F), lambda i: (0, 0, 0)),
        ],
        out_specs=pl.BlockSpec((TB, _F, _P), lambda i: (i, 0, 0)),
    )(sc_out, board3, h)


def kernel(sparse_feature_input, board_input, sparse_feature_dim,
           pcode_embedding, pcode_outerboard_embedding, offset_map):
    valid = jnp.all(sparse_feature_dim[:, 10:12] == _PCODE_DIM)
    pc = sparse_feature_input[:, 10:12].reshape(_BATCH, 2, _P)
    bd = board_input.reshape(_BATCH, 2, _P)

    npad = _PP - _P
    pad_pc = ((jnp.arange(npad, dtype=jnp.int32) * 97) % _PCODE_DIM)
    pad_pc = jnp.broadcast_to(pad_pc, (_BATCH, 2, npad))
    pad_bd = jnp.ones((_BATCH, 2, npad), jnp.int32)
    inp = jnp.concatenate(
        [jnp.concatenate([pc, pad_pc], axis=2),
         jnp.concatenate([bd, pad_bd], axis=2)], axis=1)

    off_flat = offset_map.reshape(_P)
    pad_off = ((jnp.arange(npad, dtype=jnp.int32) * 31) % 121) * _EMBED_DIM
    offp = jnp.concatenate([off_flat, pad_off])
    valid16 = jnp.full((16,), valid.astype(jnp.int32), dtype=jnp.int32)

    sent = jnp.concatenate([
        jnp.full((_P,), _PCODE_DIM, jnp.int32),
        jnp.full((_P,), 2 * _PCODE_DIM + 1, jnp.int32),
    ])
    hpad = _HROWS - 2 * _P
    hs_idx = jnp.concatenate([sent, (jnp.arange(hpad, dtype=jnp.int32) * 13) % _PCODE_DIM])
    hb_idx = jnp.concatenate([
        sent + jnp.concatenate([off_flat, off_flat]),
        (jnp.arange(hpad, dtype=jnp.int32) * 17) % _EMBED_DIM,
    ])

    sc_out, h_out = _sc_embed(inp, offp, valid16, hs_idx, hb_idx,
                              pcode_embedding, pcode_outerboard_embedding)
    h = h_out[: 2 * _P].reshape(2, _P, _F)
    out = _tc_finish(sc_out, bd, h)
    return out.reshape(_BATCH, _F, _BOARD, _BOARD)


# speedup vs baseline: 1.0941x; 1.0269x over previous
import functools

import jax
import jax.numpy as jnp
from jax import lax
from jax.experimental import pallas as pl
from jax.experimental.pallas import tpu as pltpu
from jax.experimental.pallas import tpu_sc as plsc

_F = 128
_BOARD = 15
_P = _BOARD * _BOARD
_PP = 256
_PCODE_DIM = 2380
_EMBED_DIM = 2 * (_PCODE_DIM + 1)
_BATCH = 1024
_NTILES = 32
_BPT = _BATCH // _NTILES
_HROWS = 464


def _sc_embed(inp, offp, valid16, hs_idx, hb_idx, tbl_small, tbl_big):
    mesh = plsc.VectorSubcoreMesh(
        core_axis_name="c", subcore_axis_name="s", num_cores=2, num_subcores=16
    )

    @functools.partial(
        pl.kernel,
        out_type=(
            jax.ShapeDtypeStruct((_BATCH, _P, _F), jnp.float32),
            jax.ShapeDtypeStruct((_HROWS, _F), jnp.float32),
        ),
        mesh=mesh,
        scratch_types=[
            pltpu.VMEM((4, _PP), jnp.int32),
            pltpu.VMEM((_PP,), jnp.int32),
            pltpu.VMEM((16,), jnp.int32),
            pltpu.VMEM((_PP,), jnp.int32),
            pltpu.VMEM((_PP,), jnp.int32),
            pltpu.VMEM((_PP,), jnp.int32),
            pltpu.VMEM((_PP,), jnp.int32),
            pltpu.VMEM((2, _PP, 16), jnp.float32),
            pltpu.VMEM((128, _F), jnp.float32),
            pltpu.VMEM((64, _F), jnp.float32),
            pltpu.VMEM((64, _F), jnp.float32),
            pltpu.VMEM((64, _F), jnp.float32),
            pltpu.VMEM((64, _F), jnp.float32),
            pltpu.VMEM((16,), jnp.int32),
            pltpu.VMEM((16,), jnp.int32),
            pltpu.SemaphoreType.DMA,
            pltpu.SemaphoreType.DMA,
        ],
    )
    def k(inp_h, offp_h, valid_h, hsi_h, hbi_h, tbls_h, tblb_h, out_h, hout_h,
          ibuf, offb, vb, i0e, i0o, i1e, i1o, wexp, acc,
          gA0, gB0, gA1, gB1, hsb, hbb, semG, semW):
        wid = lax.axis_index("s") * 2 + lax.axis_index("c")
        pltpu.sync_copy(offp_h, offb)
        pltpu.sync_copy(valid_h, vb)

        @pl.when(wid < _HROWS // 16)
        def _h_phase():
            pltpu.sync_copy(hsi_h.at[pl.ds(wid * 16, 16)], hsb)
            pltpu.sync_copy(hbi_h.at[pl.ds(wid * 16, 16)], hbb)
            ha = pltpu.async_copy(tbls_h.at[hsb], gA0.at[pl.ds(0, 16)], semG)
            hb = pltpu.async_copy(tblb_h.at[hbb], gB0.at[pl.ds(0, 16)], semG)
            ha.wait()
            hb.wait()
            for r in range(16):
                for kk in range(_F // 16):
                    sl = pl.ds(kk * 16, 16)
                    gA0[r, sl] = gA0[r, sl] + gB0[r, sl]
            pltpu.sync_copy(gA0.at[pl.ds(0, 16)], hout_h.at[pl.ds(wid * 16, 16)])

        gbufs = ((gA0, gB0), (gA1, gB1))
        idx_e = (i0e, i1e)
        idx_o = (i0o, i1o)

        def fire(c, half, q, s):
            sl = pl.ds(half * 128 + q * 64, 64)
            gA, gB = gbufs[s]
            return (pltpu.async_copy(tbls_h.at[idx_e[c].at[sl]], gA, semG),
                    pltpu.async_copy(tblb_h.at[idx_o[c].at[sl]], gB, semG))

        def accum(c, half, q, s):
            gA, gB = gbufs[s]

            def row(r, _):
                w = wexp[c, half * 128 + q * 64 + r, :]
                for kk in range(_F // 16):
                    sl = pl.ds(kk * 16, 16)
                    v = (gA[r, sl] + gB[r, sl]) * w
                    if c == 0:
                        acc[q * 64 + r, sl] = v
                    else:
                        plsc.addupdate(acc.at[q * 64 + r, sl], v)
                return 0

            lax.fori_loop(0, 64, row, 0)

        def per_b(j, _):
            b = wid * _BPT + j
            pltpu.sync_copy(inp_h.at[b], ibuf)
            vv = vb[...]
            for i in range(_PP // 16):
                sl = pl.ds(i * 16, 16)
                offv = offb[sl]
                pc0 = ibuf[0, sl]
                pc1 = ibuf[1, sl]
                e0 = pc0 * vv
                e1 = pc1 * vv + (_PCODE_DIM + 1)
                i0e[sl] = e0
                i0o[sl] = e0 + offv
                i1e[sl] = e1
                i1o[sl] = e1 + offv

            def build_w():
                one = jnp.full((16,), 1.0, dtype=jnp.float32)
                zero = jnp.full((16,), 0.0, dtype=jnp.float32)
                for i in range(_PP // 16):
                    sl = pl.ds(i * 16, 16)
                    w0 = jnp.where(ibuf[2, sl] > 0, zero, one)
                    w1 = jnp.where(ibuf[3, sl] > 0, zero, one)
                    for l in range(16):
                        wexp[0, i * 16 + l, :] = jnp.full((16,), w0[l], dtype=jnp.float32)
                        wexp[1, i * 16 + l, :] = jnp.full((16,), w1[l], dtype=jnp.float32)

            first = [True]
            pw = [None]
            for half in range(2):
                units = [(c, q) for c in range(2) for q in range(2)]
                pend = [None, None]
                first_accum = [True]
                for u, (c, q) in enumerate(units):
                    s = u % 2
                    if pend[s] is not None:
                        pc_, pq_, hh_ = pend[s]
                        for hh in hh_:
                            hh.wait()
                        if first_accum[0]:
                            if pw[0] is not None:
                                pw[0].wait()
                                pw[0] = None
                            first_accum[0] = False
                        accum(pc_, pq_[0], pq_[1], s)
                    pend[s] = (c, (half, q), fire(c, half, q, s))
                    if first[0] and u == 1:
                        build_w()
                        first[0] = False
                for s in range(2):
                    pc_, pq_, hh_ = pend[s]
                    for hh in hh_:
                        hh.wait()
                    accum(pc_, pq_[0], pq_[1], s)
                if half == 0:
                    pw[0] = pltpu.async_copy(
                        acc.at[pl.ds(0, 128)],
                        out_h.at[b].at[pl.ds(0, 128)], semW)
                else:
                    pltpu.sync_copy(acc.at[pl.ds(0, _P - 128)],
                                    out_h.at[b].at[pl.ds(128, _P - 128)])
            return 0

        lax.fori_loop(0, _BPT, per_b, 0)

    return k(inp, offp, valid16, hs_idx, hb_idx, tbl_small, tbl_big)


def _tc_finish(sc_out, board3, h):
    TB = 8

    def body(sc_ref, bd_ref, h_ref, o_ref):
        x = sc_ref[...]
        bd = bd_ref[...]
        hh = h_ref[...]
        m0 = (bd[:, 0, :] > 0).astype(jnp.float32)[..., None]
        m1 = (bd[:, 1, :] > 0).astype(jnp.float32)[..., None]
        y = x + m0 * hh[0] + m1 * hh[1]
        o_ref[...] = jnp.transpose(y, (0, 2, 1))

    return pl.pallas_call(
        body,
        out_shape=jax.ShapeDtypeStruct((_BATCH, _F, _P), jnp.float32),
        grid=(_BATCH // TB,),
        in_specs=[
            pl.BlockSpec((TB, _P, _F), lambda i: (i, 0, 0)),
            pl.BlockSpec((TB, 2, _P), lambda i: (i, 0, 0)),
            pl.BlockSpec((2, _P, _F), lambda i: (0, 0, 0)),
        ],
        out_specs=pl.BlockSpec((TB, _F, _P), lambda i: (i, 0, 0)),
    )(sc_out, board3, h)


def kernel(sparse_feature_input, board_input, sparse_feature_dim,
           pcode_embedding, pcode_outerboard_embedding, offset_map):
    valid = jnp.all(sparse_feature_dim[:, 10:12] == _PCODE_DIM)
    pc = sparse_feature_input[:, 10:12].reshape(_BATCH, 2, _P)
    bd = board_input.reshape(_BATCH, 2, _P)

    npad = _PP - _P
    pad_pc = ((jnp.arange(npad, dtype=jnp.int32) * 97) % _PCODE_DIM)
    pad_pc = jnp.broadcast_to(pad_pc, (_BATCH, 2, npad))
    pad_bd = jnp.ones((_BATCH, 2, npad), jnp.int32)
    inp = jnp.concatenate(
        [jnp.concatenate([pc, pad_pc], axis=2),
         jnp.concatenate([bd, pad_bd], axis=2)], axis=1)

    off_flat = offset_map.reshape(_P)
    pad_off = ((jnp.arange(npad, dtype=jnp.int32) * 31) % 121) * _EMBED_DIM
    offp = jnp.concatenate([off_flat, pad_off])
    valid16 = jnp.full((16,), valid.astype(jnp.int32), dtype=jnp.int32)

    sent = jnp.concatenate([
        jnp.full((_P,), _PCODE_DIM, jnp.int32),
        jnp.full((_P,), 2 * _PCODE_DIM + 1, jnp.int32),
    ])
    hpad = _HROWS - 2 * _P
    hs_idx = jnp.concatenate([sent, (jnp.arange(hpad, dtype=jnp.int32) * 13) % _PCODE_DIM])
    hb_idx = jnp.concatenate([
        sent + jnp.concatenate([off_flat, off_flat]),
        (jnp.arange(hpad, dtype=jnp.int32) * 17) % _EMBED_DIM,
    ])

    sc_out, h_out = _sc_embed(inp, offp, valid16, hs_idx, hb_idx,
                              pcode_embedding, pcode_outerboard_embedding)
    h = h_out[: 2 * _P].reshape(2, _P, _F)
    out = _tc_finish(sc_out, bd, h)
    return out.reshape(_BATCH, _F, _BOARD, _BOARD)


# speedup vs baseline: 1.1482x; 1.0494x over previous
import functools

import jax
import jax.numpy as jnp
from jax import lax
from jax.experimental import pallas as pl
from jax.experimental.pallas import tpu as pltpu
from jax.experimental.pallas import tpu_sc as plsc

_F = 128
_BOARD = 15
_P = _BOARD * _BOARD
_PP = 256
_PCODE_DIM = 2380
_EMBED_DIM = 2 * (_PCODE_DIM + 1)
_BATCH = 1024
_NTILES = 32
_BPT = _BATCH // _NTILES
_HROWS = 464


def _sc_embed(inp, offp, valid16, hs_idx, hb_idx, tbl_small, tbl_big):
    mesh = plsc.VectorSubcoreMesh(
        core_axis_name="c", subcore_axis_name="s", num_cores=2, num_subcores=16
    )

    @functools.partial(
        pl.kernel,
        out_type=(
            jax.ShapeDtypeStruct((_BATCH, _P, _F), jnp.float32),
            jax.ShapeDtypeStruct((_HROWS, _F), jnp.float32),
        ),
        mesh=mesh,
        scratch_types=[
            pltpu.VMEM((4, _PP), jnp.int32),
            pltpu.VMEM((_PP,), jnp.int32),
            pltpu.VMEM((16,), jnp.int32),
            pltpu.VMEM((_PP,), jnp.int32),
            pltpu.VMEM((_PP,), jnp.int32),
            pltpu.VMEM((_PP,), jnp.int32),
            pltpu.VMEM((_PP,), jnp.int32),
            pltpu.VMEM((2, _PP, 16), jnp.float32),
            pltpu.VMEM((128, _F), jnp.float32),
            pltpu.VMEM((64, _F), jnp.float32),
            pltpu.VMEM((64, _F), jnp.float32),
            pltpu.VMEM((64, _F), jnp.float32),
            pltpu.VMEM((64, _F), jnp.float32),
            pltpu.VMEM((16,), jnp.int32),
            pltpu.VMEM((16,), jnp.int32),
            pltpu.SemaphoreType.DMA,
            pltpu.SemaphoreType.DMA,
        ],
    )
    def k(inp_h, offp_h, valid_h, hsi_h, hbi_h, tbls_h, tblb_h, out_h, hout_h,
          ibuf, offb, vb, i0e, i0o, i1e, i1o, wexp, acc,
          gA0, gB0, gA1, gB1, hsb, hbb, semG, semW):
        wid = lax.axis_index("s") * 2 + lax.axis_index("c")
        pltpu.sync_copy(offp_h, offb)
        pltpu.sync_copy(valid_h, vb)

        @pl.when(wid < _HROWS // 16)
        def _h_phase():
            pltpu.sync_copy(hsi_h.at[pl.ds(wid * 16, 16)], hsb)
            pltpu.sync_copy(hbi_h.at[pl.ds(wid * 16, 16)], hbb)
            ha = pltpu.async_copy(tbls_h.at[hsb], gA0.at[pl.ds(0, 16)], semG)
            hb = pltpu.async_copy(tblb_h.at[hbb], gB0.at[pl.ds(0, 16)], semG)
            ha.wait()
            hb.wait()
            for r in range(16):
                for kk in range(_F // 16):
                    sl = pl.ds(kk * 16, 16)
                    gA0[r, sl] = gA0[r, sl] + gB0[r, sl]
            pltpu.sync_copy(gA0.at[pl.ds(0, 16)], hout_h.at[pl.ds(wid * 16, 16)])

        gbufs = ((gA0, gB0), (gA1, gB1))
        idx_e = (i0e, i1e)
        idx_o = (i0o, i1o)

        def fire(c, half, q, s):
            sl = pl.ds(half * 128 + q * 64, 64)
            gA, gB = gbufs[s]
            return (pltpu.async_copy(tbls_h.at[idx_e[c].at[sl]], gA, semG),
                    pltpu.async_copy(tblb_h.at[idx_o[c].at[sl]], gB, semG))

        def accum(c, half, q, s):
            gA, gB = gbufs[s]

            def row(r, _):
                w = wexp[c, half * 128 + q * 64 + r, :]
                for kk in range(_F // 16):
                    sl = pl.ds(kk * 16, 16)
                    v = (gA[r, sl] + gB[r, sl]) * w
                    if c == 0:
                        acc[q * 64 + r, sl] = v
                    else:
                        plsc.addupdate(acc.at[q * 64 + r, sl], v)
                return 0

            lax.fori_loop(0, 64, row, 0)

        def per_b(j, _):
            b = wid * _BPT + j
            pltpu.sync_copy(inp_h.at[b], ibuf)
            vv = vb[...]
            for i in range(_PP // 16):
                sl = pl.ds(i * 16, 16)
                offv = offb[sl]
                pc0 = ibuf[0, sl]
                pc1 = ibuf[1, sl]
                e0 = pc0 * vv
                e1 = pc1 * vv + (_PCODE_DIM + 1)
                i0e[sl] = e0
                i0o[sl] = e0 + offv
                i1e[sl] = e1
                i1o[sl] = e1 + offv

            def build_w():
                one = jnp.full((16,), 1.0, dtype=jnp.float32)
                zero = jnp.full((16,), 0.0, dtype=jnp.float32)
                for i in range(_PP // 16):
                    sl = pl.ds(i * 16, 16)
                    w0 = jnp.where(ibuf[2, sl] > 0, zero, one)
                    w1 = jnp.where(ibuf[3, sl] > 0, zero, one)
                    for l in range(16):
                        wexp[0, i * 16 + l, :] = jnp.full((16,), w0[l], dtype=jnp.float32)
                        wexp[1, i * 16 + l, :] = jnp.full((16,), w1[l], dtype=jnp.float32)

            first = [True]
            pw = [None]
            for half in range(2):
                units = [(c, q) for c in range(2) for q in range(2)]
                pend = [None, None]
                first_accum = [True]
                for u, (c, q) in enumerate(units):
                    s = u % 2
                    if pend[s] is not None:
                        pc_, pq_, hh_ = pend[s]
                        for hh in hh_:
                            hh.wait()
                        if first_accum[0]:
                            if pw[0] is not None:
                                pw[0].wait()
                                pw[0] = None
                            first_accum[0] = False
                        accum(pc_, pq_[0], pq_[1], s)
                    pend[s] = (c, (half, q), fire(c, half, q, s))
                    if first[0] and u == 1:
                        build_w()
                        first[0] = False
                for s in range(2):
                    pc_, pq_, hh_ = pend[s]
                    for hh in hh_:
                        hh.wait()
                    accum(pc_, pq_[0], pq_[1], s)
                if half == 0:
                    pw[0] = pltpu.async_copy(
                        acc.at[pl.ds(0, 128)],
                        out_h.at[b].at[pl.ds(0, 128)], semW)
                else:
                    pltpu.sync_copy(acc.at[pl.ds(0, _P - 128)],
                                    out_h.at[b].at[pl.ds(128, _P - 128)])
            return 0

        lax.fori_loop(0, _BPT, per_b, 0)

    return k(inp, offp, valid16, hs_idx, hb_idx, tbl_small, tbl_big)


def _tc_finish(sc_out, board3, h):
    TB = 16

    def body(sc_ref, bd_ref, h_ref, o_ref):
        x = sc_ref[...]
        bd = bd_ref[...]
        hh = h_ref[...]
        m0 = (bd[:, 0, :] > 0).astype(jnp.float32)[..., None]
        m1 = (bd[:, 1, :] > 0).astype(jnp.float32)[..., None]
        y = x + m0 * hh[0] + m1 * hh[1]
        o_ref[...] = jnp.transpose(y, (0, 2, 1))

    return pl.pallas_call(
        body,
        out_shape=jax.ShapeDtypeStruct((_BATCH, _F, _P), jnp.float32),
        grid=(_BATCH // TB,),
        in_specs=[
            pl.BlockSpec((TB, _P, _F), lambda i: (i, 0, 0)),
            pl.BlockSpec((TB, 2, _P), lambda i: (i, 0, 0)),
            pl.BlockSpec((2, _P, _F), lambda i: (0, 0, 0)),
        ],
        out_specs=pl.BlockSpec((TB, _F, _P), lambda i: (i, 0, 0)),
    )(sc_out, board3, h)


def kernel(sparse_feature_input, board_input, sparse_feature_dim,
           pcode_embedding, pcode_outerboard_embedding, offset_map):
    valid = jnp.all(sparse_feature_dim[:, 10:12] == _PCODE_DIM)
    pc = sparse_feature_input[:, 10:12].reshape(_BATCH, 2, _P)
    bd = board_input.reshape(_BATCH, 2, _P)

    npad = _PP - _P
    pad_pc = ((jnp.arange(npad, dtype=jnp.int32) * 97) % _PCODE_DIM)
    pad_pc = jnp.broadcast_to(pad_pc, (_BATCH, 2, npad))
    pad_bd = jnp.ones((_BATCH, 2, npad), jnp.int32)
    inp = jnp.concatenate(
        [jnp.concatenate([pc, pad_pc], axis=2),
         jnp.concatenate([bd, pad_bd], axis=2)], axis=1)

    off_flat = offset_map.reshape(_P)
    pad_off = ((jnp.arange(npad, dtype=jnp.int32) * 31) % 121) * _EMBED_DIM
    offp = jnp.concatenate([off_flat, pad_off])
    valid16 = jnp.full((16,), valid.astype(jnp.int32), dtype=jnp.int32)

    sent = jnp.concatenate([
        jnp.full((_P,), _PCODE_DIM, jnp.int32),
        jnp.full((_P,), 2 * _PCODE_DIM + 1, jnp.int32),
    ])
    hpad = _HROWS - 2 * _P
    hs_idx = jnp.concatenate([sent, (jnp.arange(hpad, dtype=jnp.int32) * 13) % _PCODE_DIM])
    hb_idx = jnp.concatenate([
        sent + jnp.concatenate([off_flat, off_flat]),
        (jnp.arange(hpad, dtype=jnp.int32) * 17) % _EMBED_DIM,
    ])

    sc_out, h_out = _sc_embed(inp, offp, valid16, hs_idx, hb_idx,
                              pcode_embedding, pcode_outerboard_embedding)
    h = h_out[: 2 * _P].reshape(2, _P, _F)
    out = _tc_finish(sc_out, bd, h)
    return out.reshape(_BATCH, _F, _BOARD, _BOARD)


# speedup vs baseline: 1.1741x; 1.0225x over previous
import functools

import jax
import jax.numpy as jnp
from jax import lax
from jax.experimental import pallas as pl
from jax.experimental.pallas import tpu as pltpu
from jax.experimental.pallas import tpu_sc as plsc

_F = 128
_BOARD = 15
_P = _BOARD * _BOARD
_PP = 256
_PCODE_DIM = 2380
_EMBED_DIM = 2 * (_PCODE_DIM + 1)
_BATCH = 1024
_NTILES = 32
_BPT = _BATCH // _NTILES
_HROWS = 464


def _sc_embed(inp, offp, valid16, hs_idx, hb_idx, tbl_small, tbl_big):
    mesh = plsc.VectorSubcoreMesh(
        core_axis_name="c", subcore_axis_name="s", num_cores=2, num_subcores=16
    )

    @functools.partial(
        pl.kernel,
        out_type=(
            jax.ShapeDtypeStruct((_BATCH, _P, _F), jnp.float32),
            jax.ShapeDtypeStruct((_HROWS, _F), jnp.float32),
        ),
        mesh=mesh,
        scratch_types=[
            pltpu.VMEM((4, _PP), jnp.int32),
            pltpu.VMEM((_PP,), jnp.int32),
            pltpu.VMEM((16,), jnp.int32),
            pltpu.VMEM((_PP,), jnp.int32),
            pltpu.VMEM((_PP,), jnp.int32),
            pltpu.VMEM((_PP,), jnp.int32),
            pltpu.VMEM((_PP,), jnp.int32),
            pltpu.VMEM((2, _PP, 16), jnp.float32),
            pltpu.VMEM((128, _F), jnp.float32),
            pltpu.VMEM((64, _F), jnp.float32),
            pltpu.VMEM((64, _F), jnp.float32),
            pltpu.VMEM((64, _F), jnp.float32),
            pltpu.VMEM((64, _F), jnp.float32),
            pltpu.VMEM((16,), jnp.int32),
            pltpu.VMEM((16,), jnp.int32),
            pltpu.SemaphoreType.DMA,
            pltpu.SemaphoreType.DMA,
        ],
    )
    def k(inp_h, offp_h, valid_h, hsi_h, hbi_h, tbls_h, tblb_h, out_h, hout_h,
          ibuf, offb, vb, i0e, i0o, i1e, i1o, wexp, acc,
          gA0, gB0, gA1, gB1, hsb, hbb, semG, semW):
        wid = lax.axis_index("s") * 2 + lax.axis_index("c")
        pltpu.sync_copy(offp_h, offb)
        pltpu.sync_copy(valid_h, vb)

        @pl.when(wid < _HROWS // 16)
        def _h_phase():
            pltpu.sync_copy(hsi_h.at[pl.ds(wid * 16, 16)], hsb)
            pltpu.sync_copy(hbi_h.at[pl.ds(wid * 16, 16)], hbb)
            ha = pltpu.async_copy(tbls_h.at[hsb], gA0.at[pl.ds(0, 16)], semG)
            hb = pltpu.async_copy(tblb_h.at[hbb], gB0.at[pl.ds(0, 16)], semG)
            ha.wait()
            hb.wait()
            for r in range(16):
                for kk in range(_F // 16):
                    sl = pl.ds(kk * 16, 16)
                    gA0[r, sl] = gA0[r, sl] + gB0[r, sl]
            pltpu.sync_copy(gA0.at[pl.ds(0, 16)], hout_h.at[pl.ds(wid * 16, 16)])

        gbufs = ((gA0, gB0), (gA1, gB1))
        idx_e = (i0e, i1e)
        idx_o = (i0o, i1o)

        def fire(c, half, q, s):
            sl = pl.ds(half * 128 + q * 64, 64)
            gA, gB = gbufs[s]
            return (pltpu.async_copy(tbls_h.at[idx_e[c].at[sl]], gA, semG),
                    pltpu.async_copy(tblb_h.at[idx_o[c].at[sl]], gB, semG))

        def accum(c, half, q, s):
            gA, gB = gbufs[s]

            def row(r, _):
                w = wexp[c, half * 128 + q * 64 + r, :]
                for kk in range(_F // 16):
                    sl = pl.ds(kk * 16, 16)
                    v = (gA[r, sl] + gB[r, sl]) * w
                    if c == 0:
                        acc[q * 64 + r, sl] = v
                    else:
                        plsc.addupdate(acc.at[q * 64 + r, sl], v)
                return 0

            lax.fori_loop(0, 64, row, 0)

        def per_b(j, _):
            b = wid * _BPT + j
            pltpu.sync_copy(inp_h.at[b], ibuf)
            vv = vb[...]
            for i in range(_PP // 16):
                sl = pl.ds(i * 16, 16)
                offv = offb[sl]
                pc0 = ibuf[0, sl]
                pc1 = ibuf[1, sl]
                e0 = pc0 * vv
                e1 = pc1 * vv + (_PCODE_DIM + 1)
                i0e[sl] = e0
                i0o[sl] = e0 + offv
                i1e[sl] = e1
                i1o[sl] = e1 + offv

            def build_w():
                one = jnp.full((16,), 1.0, dtype=jnp.float32)
                zero = jnp.full((16,), 0.0, dtype=jnp.float32)
                for i in range(_PP // 16):
                    sl = pl.ds(i * 16, 16)
                    w0 = jnp.where(ibuf[2, sl] > 0, zero, one)
                    w1 = jnp.where(ibuf[3, sl] > 0, zero, one)
                    for l in range(16):
                        wexp[0, i * 16 + l, :] = jnp.full((16,), w0[l], dtype=jnp.float32)
                        wexp[1, i * 16 + l, :] = jnp.full((16,), w1[l], dtype=jnp.float32)

            first = [True]
            pw = [None]
            for half in range(2):
                units = [(c, q) for c in range(2) for q in range(2)]
                pend = [None, None]
                first_accum = [True]
                for u, (c, q) in enumerate(units):
                    s = u % 2
                    if pend[s] is not None:
                        pc_, pq_, hh_ = pend[s]
                        for hh in hh_:
                            hh.wait()
                        if first_accum[0]:
                            if pw[0] is not None:
                                pw[0].wait()
                                pw[0] = None
                            first_accum[0] = False
                        accum(pc_, pq_[0], pq_[1], s)
                    pend[s] = (c, (half, q), fire(c, half, q, s))
                    if first[0] and u == 1:
                        build_w()
                        first[0] = False
                for s in range(2):
                    pc_, pq_, hh_ = pend[s]
                    for hh in hh_:
                        hh.wait()
                    accum(pc_, pq_[0], pq_[1], s)
                if half == 0:
                    pw[0] = pltpu.async_copy(
                        acc.at[pl.ds(0, 128)],
                        out_h.at[b].at[pl.ds(0, 128)], semW)
                else:
                    pltpu.sync_copy(acc.at[pl.ds(0, _P - 128)],
                                    out_h.at[b].at[pl.ds(128, _P - 128)])
            return 0

        lax.fori_loop(0, _BPT, per_b, 0)

    return k(inp, offp, valid16, hs_idx, hb_idx, tbl_small, tbl_big)


def _tc_finish(sc_out, board3, h):
    TB = 32

    def body(sc_ref, bd_ref, h_ref, o_ref):
        x = sc_ref[...]
        bd = bd_ref[...]
        hh = h_ref[...]
        m0 = (bd[:, 0, :] > 0).astype(jnp.float32)[..., None]
        m1 = (bd[:, 1, :] > 0).astype(jnp.float32)[..., None]
        y = x + m0 * hh[0] + m1 * hh[1]
        o_ref[...] = jnp.transpose(y, (0, 2, 1))

    return pl.pallas_call(
        body,
        out_shape=jax.ShapeDtypeStruct((_BATCH, _F, _P), jnp.float32),
        grid=(_BATCH // TB,),
        in_specs=[
            pl.BlockSpec((TB, _P, _F), lambda i: (i, 0, 0)),
            pl.BlockSpec((TB, 2, _P), lambda i: (i, 0, 0)),
            pl.BlockSpec((2, _P, _F), lambda i: (0, 0, 0)),
        ],
        out_specs=pl.BlockSpec((TB, _F, _P), lambda i: (i, 0, 0)),
    )(sc_out, board3, h)


def kernel(sparse_feature_input, board_input, sparse_feature_dim,
           pcode_embedding, pcode_outerboard_embedding, offset_map):
    valid = jnp.all(sparse_feature_dim[:, 10:12] == _PCODE_DIM)
    pc = sparse_feature_input[:, 10:12].reshape(_BATCH, 2, _P)
    bd = board_input.reshape(_BATCH, 2, _P)

    npad = _PP - _P
    pad_pc = ((jnp.arange(npad, dtype=jnp.int32) * 97) % _PCODE_DIM)
    pad_pc = jnp.broadcast_to(pad_pc, (_BATCH, 2, npad))
    pad_bd = jnp.ones((_BATCH, 2, npad), jnp.int32)
    inp = jnp.concatenate(
        [jnp.concatenate([pc, pad_pc], axis=2),
         jnp.concatenate([bd, pad_bd], axis=2)], axis=1)

    off_flat = offset_map.reshape(_P)
    pad_off = ((jnp.arange(npad, dtype=jnp.int32) * 31) % 121) * _EMBED_DIM
    offp = jnp.concatenate([off_flat, pad_off])
    valid16 = jnp.full((16,), valid.astype(jnp.int32), dtype=jnp.int32)

    sent = jnp.concatenate([
        jnp.full((_P,), _PCODE_DIM, jnp.int32),
        jnp.full((_P,), 2 * _PCODE_DIM + 1, jnp.int32),
    ])
    hpad = _HROWS - 2 * _P
    hs_idx = jnp.concatenate([sent, (jnp.arange(hpad, dtype=jnp.int32) * 13) % _PCODE_DIM])
    hb_idx = jnp.concatenate([
        sent + jnp.concatenate([off_flat, off_flat]),
        (jnp.arange(hpad, dtype=jnp.int32) * 17) % _EMBED_DIM,
    ])

    sc_out, h_out = _sc_embed(inp, offp, valid16, hs_idx, hb_idx,
                              pcode_embedding, pcode_outerboard_embedding)
    h = h_out[: 2 * _P].reshape(2, _P, _F)
    out = _tc_finish(sc_out, bd, h)
    return out.reshape(_BATCH, _F, _BOARD, _BOARD)


# speedup vs baseline: 1.1873x; 1.0113x over previous
import functools

import jax
import jax.numpy as jnp
from jax import lax
from jax.experimental import pallas as pl
from jax.experimental.pallas import tpu as pltpu
from jax.experimental.pallas import tpu_sc as plsc

_F = 128
_BOARD = 15
_P = _BOARD * _BOARD
_PP = 256
_PCODE_DIM = 2380
_EMBED_DIM = 2 * (_PCODE_DIM + 1)
_BATCH = 1024
_NTILES = 32
_BPT = _BATCH // _NTILES
_HROWS = 464


def _sc_embed(inp, offp, valid16, hs_idx, hb_idx, tbl_small, tbl_big):
    mesh = plsc.VectorSubcoreMesh(
        core_axis_name="c", subcore_axis_name="s", num_cores=2, num_subcores=16
    )

    @functools.partial(
        pl.kernel,
        out_type=(
            jax.ShapeDtypeStruct((_BATCH, _P, _F), jnp.float32),
            jax.ShapeDtypeStruct((_HROWS, _F), jnp.float32),
        ),
        mesh=mesh,
        scratch_types=[
            pltpu.VMEM((4, _PP), jnp.int32),
            pltpu.VMEM((_PP,), jnp.int32),
            pltpu.VMEM((16,), jnp.int32),
            pltpu.VMEM((_PP,), jnp.int32),
            pltpu.VMEM((_PP,), jnp.int32),
            pltpu.VMEM((_PP,), jnp.int32),
            pltpu.VMEM((_PP,), jnp.int32),
            pltpu.VMEM((2, _PP, 16), jnp.float32),
            pltpu.VMEM((128, _F), jnp.float32),
            pltpu.VMEM((64, _F), jnp.float32),
            pltpu.VMEM((64, _F), jnp.float32),
            pltpu.VMEM((64, _F), jnp.float32),
            pltpu.VMEM((64, _F), jnp.float32),
            pltpu.VMEM((16,), jnp.int32),
            pltpu.VMEM((16,), jnp.int32),
            pltpu.SemaphoreType.DMA,
            pltpu.SemaphoreType.DMA,
        ],
    )
    def k(inp_h, offp_h, valid_h, hsi_h, hbi_h, tbls_h, tblb_h, out_h, hout_h,
          ibuf, offb, vb, i0e, i0o, i1e, i1o, wexp, acc,
          gA0, gB0, gA1, gB1, hsb, hbb, semG, semW):
        wid = lax.axis_index("s") * 2 + lax.axis_index("c")
        pltpu.sync_copy(offp_h, offb)
        pltpu.sync_copy(valid_h, vb)

        @pl.when(wid < _HROWS // 16)
        def _h_phase():
            pltpu.sync_copy(hsi_h.at[pl.ds(wid * 16, 16)], hsb)
            pltpu.sync_copy(hbi_h.at[pl.ds(wid * 16, 16)], hbb)
            ha = pltpu.async_copy(tbls_h.at[hsb], gA0.at[pl.ds(0, 16)], semG)
            hb = pltpu.async_copy(tblb_h.at[hbb], gB0.at[pl.ds(0, 16)], semG)
            ha.wait()
            hb.wait()
            for r in range(16):
                for kk in range(_F // 16):
                    sl = pl.ds(kk * 16, 16)
                    gA0[r, sl] = gA0[r, sl] + gB0[r, sl]
            pltpu.sync_copy(gA0.at[pl.ds(0, 16)], hout_h.at[pl.ds(wid * 16, 16)])

        gbufs = ((gA0, gB0), (gA1, gB1))
        idx_e = (i0e, i1e)
        idx_o = (i0o, i1o)

        def fire(c, half, q, s):
            sl = pl.ds(half * 128 + q * 64, 64)
            gA, gB = gbufs[s]
            return (pltpu.async_copy(tbls_h.at[idx_e[c].at[sl]], gA, semG),
                    pltpu.async_copy(tblb_h.at[idx_o[c].at[sl]], gB, semG))

        def accum(c, half, q, s):
            gA, gB = gbufs[s]

            def row(r, _):
                w = wexp[c, half * 128 + q * 64 + r, :]
                for kk in range(_F // 16):
                    sl = pl.ds(kk * 16, 16)
                    v = (gA[r, sl] + gB[r, sl]) * w
                    if c == 0:
                        acc[q * 64 + r, sl] = v
                    else:
                        plsc.addupdate(acc.at[q * 64 + r, sl], v)
                return 0

            lax.fori_loop(0, 64, row, 0)

        def per_b(j, _):
            b = wid * _BPT + j
            pltpu.sync_copy(inp_h.at[b], ibuf)
            vv = vb[...]
            for i in range(_PP // 16):
                sl = pl.ds(i * 16, 16)
                offv = offb[sl]
                pc0 = ibuf[0, sl]
                pc1 = ibuf[1, sl]
                e0 = pc0 * vv
                e1 = pc1 * vv + (_PCODE_DIM + 1)
                i0e[sl] = e0
                i0o[sl] = e0 + offv
                i1e[sl] = e1
                i1o[sl] = e1 + offv

            def build_w():
                one = jnp.full((16,), 1.0, dtype=jnp.float32)
                zero = jnp.full((16,), 0.0, dtype=jnp.float32)
                for i in range(_PP // 16):
                    sl = pl.ds(i * 16, 16)
                    w0 = jnp.where(ibuf[2, sl] > 0, zero, one)
                    w1 = jnp.where(ibuf[3, sl] > 0, zero, one)
                    for l in range(16):
                        wexp[0, i * 16 + l, :] = jnp.full((16,), w0[l], dtype=jnp.float32)
                        wexp[1, i * 16 + l, :] = jnp.full((16,), w1[l], dtype=jnp.float32)

            first = [True]
            pw = [None]
            for half in range(2):
                units = [(c, q) for c in range(2) for q in range(2)]
                pend = [None, None]
                first_accum = [True]
                for u, (c, q) in enumerate(units):
                    s = u % 2
                    if pend[s] is not None:
                        pc_, pq_, hh_ = pend[s]
                        for hh in hh_:
                            hh.wait()
                        if first_accum[0]:
                            if pw[0] is not None:
                                pw[0].wait()
                                pw[0] = None
                            first_accum[0] = False
                        accum(pc_, pq_[0], pq_[1], s)
                    pend[s] = (c, (half, q), fire(c, half, q, s))
                    if first[0] and u == 1:
                        build_w()
                        first[0] = False
                for s in range(2):
                    pc_, pq_, hh_ = pend[s]
                    for hh in hh_:
                        hh.wait()
                    accum(pc_, pq_[0], pq_[1], s)
                if half == 0:
                    pw[0] = pltpu.async_copy(
                        acc.at[pl.ds(0, 128)],
                        out_h.at[b].at[pl.ds(0, 128)], semW)
                else:
                    pltpu.sync_copy(acc.at[pl.ds(0, _P - 128)],
                                    out_h.at[b].at[pl.ds(128, _P - 128)])
            return 0

        lax.fori_loop(0, _BPT, per_b, 0)

    return k(inp, offp, valid16, hs_idx, hb_idx, tbl_small, tbl_big)


def _tc_finish(sc_out, board3, h):
    TB = 64

    def body(sc_ref, bd_ref, h_ref, o_ref):
        x = sc_ref[...]
        bd = bd_ref[...]
        hh = h_ref[...]
        m0 = (bd[:, 0, :] > 0).astype(jnp.float32)[..., None]
        m1 = (bd[:, 1, :] > 0).astype(jnp.float32)[..., None]
        y = x + m0 * hh[0] + m1 * hh[1]
        o_ref[...] = jnp.transpose(y, (0, 2, 1))

    return pl.pallas_call(
        body,
        out_shape=jax.ShapeDtypeStruct((_BATCH, _F, _P), jnp.float32),
        grid=(_BATCH // TB,),
        in_specs=[
            pl.BlockSpec((TB, _P, _F), lambda i: (i, 0, 0)),
            pl.BlockSpec((TB, 2, _P), lambda i: (i, 0, 0)),
            pl.BlockSpec((2, _P, _F), lambda i: (0, 0, 0)),
        ],
        out_specs=pl.BlockSpec((TB, _F, _P), lambda i: (i, 0, 0)),
    )(sc_out, board3, h)


def kernel(sparse_feature_input, board_input, sparse_feature_dim,
           pcode_embedding, pcode_outerboard_embedding, offset_map):
    valid = jnp.all(sparse_feature_dim[:, 10:12] == _PCODE_DIM)
    pc = sparse_feature_input[:, 10:12].reshape(_BATCH, 2, _P)
    bd = board_input.reshape(_BATCH, 2, _P)

    npad = _PP - _P
    pad_pc = ((jnp.arange(npad, dtype=jnp.int32) * 97) % _PCODE_DIM)
    pad_pc = jnp.broadcast_to(pad_pc, (_BATCH, 2, npad))
    pad_bd = jnp.ones((_BATCH, 2, npad), jnp.int32)
    inp = jnp.concatenate(
        [jnp.concatenate([pc, pad_pc], axis=2),
         jnp.concatenate([bd, pad_bd], axis=2)], axis=1)

    off_flat = offset_map.reshape(_P)
    pad_off = ((jnp.arange(npad, dtype=jnp.int32) * 31) % 121) * _EMBED_DIM
    offp = jnp.concatenate([off_flat, pad_off])
    valid16 = jnp.full((16,), valid.astype(jnp.int32), dtype=jnp.int32)

    sent = jnp.concatenate([
        jnp.full((_P,), _PCODE_DIM, jnp.int32),
        jnp.full((_P,), 2 * _PCODE_DIM + 1, jnp.int32),
    ])
    hpad = _HROWS - 2 * _P
    hs_idx = jnp.concatenate([sent, (jnp.arange(hpad, dtype=jnp.int32) * 13) % _PCODE_DIM])
    hb_idx = jnp.concatenate([
        sent + jnp.concatenate([off_flat, off_flat]),
        (jnp.arange(hpad, dtype=jnp.int32) * 17) % _EMBED_DIM,
    ])

    sc_out, h_out = _sc_embed(inp, offp, valid16, hs_idx, hb_idx,
                              pcode_embedding, pcode_outerboard_embedding)
    h = h_out[: 2 * _P].reshape(2, _P, _F)
    out = _tc_finish(sc_out, bd, h)
    return out.reshape(_BATCH, _F, _BOARD, _BOARD)
